# baseline (speedup 1.0000x reference)
import functools

import jax
import jax.numpy as jnp
from jax import lax
from jax.experimental import pallas as pl
from jax.experimental.pallas import tpu as pltpu
from jax.experimental.pallas import tpu_sc as plsc

VOCAB = 49408
EMBED = 768
NUM_POS = 77
BATCH = 4096
B = BATCH * NUM_POS
NW = 32
ROWS_PER_W = B // NW
CHUNK = 32
NCHUNK = ROWS_PER_W // CHUNK
LANES = 16
NJ = EMBED // LANES


def _sc_body(tok_hbm, idx_hbm, pos_hbm, out_hbm, pos_v, idx_v, rows_v, sem):
    c = lax.axis_index("c")
    s = lax.axis_index("s")
    wid = s * 2 + c
    base = wid * ROWS_PER_W

    pltpu.sync_copy(pos_hbm, pos_v)

    def chunk_body(k, _):
        cb = base + k * CHUNK
        pltpu.sync_copy(idx_hbm.at[pl.ds(cb, CHUNK)], idx_v)
        pltpu.async_copy(tok_hbm.at[idx_v], rows_v, sem).wait()

        def row_body(r, _):
            p = lax.rem(cb + r, NUM_POS)
            for j in range(NJ):
                sl = pl.ds(j * LANES, LANES)
                rows_v[r, sl] = rows_v[r, sl] + pos_v[p, sl]
            return 0

        lax.fori_loop(0, CHUNK, row_body, 0)
        pltpu.sync_copy(rows_v, out_hbm.at[pl.ds(cb, CHUNK)])
        return 0

    lax.fori_loop(0, NCHUNK, chunk_body, 0)


@jax.jit
def _run(idx, token_embedding, position_embedding):
    mesh = plsc.VectorSubcoreMesh(core_axis_name="c", subcore_axis_name="s")
    k = pl.kernel(
        _sc_body,
        out_type=jax.ShapeDtypeStruct((B, EMBED), jnp.float32),
        mesh=mesh,
        scratch_types=[
            pltpu.VMEM((NUM_POS, EMBED), jnp.float32),
            pltpu.VMEM((CHUNK,), jnp.int32),
            pltpu.VMEM((CHUNK, EMBED), jnp.float32),
            pltpu.SemaphoreType.DMA,
        ],
    )
    return k(token_embedding, idx, position_embedding)


def kernel(input_tokens, token_embedding, position_embedding):
    idx = input_tokens.reshape(-1).astype(jnp.int32)
    out = _run(idx, token_embedding, position_embedding)
    return out.reshape(BATCH, NUM_POS, EMBED)


# speedup vs baseline: 1.1690x; 1.1690x over previous
import jax
import jax.numpy as jnp
from jax import lax
from jax.experimental import pallas as pl
from jax.experimental.pallas import tpu as pltpu
from jax.experimental.pallas import tpu_sc as plsc

VOCAB = 49408
EMBED = 768
NUM_POS = 77
BATCH = 4096
B = BATCH * NUM_POS
NW = 32
ROWS_PER_W = B // NW
CHUNK = 32
NCHUNK = ROWS_PER_W // CHUNK
NPAIR = NCHUNK // 2
LANES = 16
NJ = EMBED // LANES


def _sc_body(tok_hbm, idx_hbm, pos_hbm, out_hbm,
             pos_v, idx_v, rows0, rows1, gsem0, gsem1, osem0, osem1):
    c = lax.axis_index("c")
    s = lax.axis_index("s")
    wid = s * 2 + c
    base = wid * ROWS_PER_W

    bufs = (rows0, rows1)
    gsems = (gsem0, gsem1)
    osems = (osem0, osem1)

    pltpu.sync_copy(pos_hbm, pos_v)
    pltpu.sync_copy(idx_hbm.at[pl.ds(base, ROWS_PER_W)], idx_v)

    def gather_start(g, b):
        pltpu.async_copy(tok_hbm.at[idx_v.at[pl.ds(g * CHUNK, CHUNK)]],
                         bufs[b], gsems[b])

    def gather_wait(g, b):
        pltpu.make_async_copy(tok_hbm.at[idx_v.at[pl.ds(g * CHUNK, CHUNK)]],
                              bufs[b], gsems[b]).wait()

    def out_start(g, b):
        pltpu.async_copy(bufs[b], out_hbm.at[pl.ds(base + g * CHUNK, CHUNK)],
                         osems[b])

    def out_drain(b):
        pltpu.make_async_copy(bufs[b], out_hbm.at[pl.ds(base, CHUNK)],
                              osems[b]).wait()

    gather_start(0, 0)

    def visit(g, b):
        bn = 1 - b
        gather_wait(g, b)

        @pl.when(g >= 1)
        def _():
            out_drain(bn)

        @pl.when(g + 1 < NCHUNK)
        def _():
            gather_start(g + 1, bn)

        p0 = lax.rem(g * CHUNK, NUM_POS)

        def row_body(r, p):
            for j in range(NJ):
                sl = pl.ds(j * LANES, LANES)
                bufs[b][r, sl] = bufs[b][r, sl] + pos_v[p, sl]
            p1 = p + 1
            return jnp.where(p1 >= NUM_POS, 0, p1)

        lax.fori_loop(0, CHUNK, row_body, p0)
        out_start(g, b)

    def pair(k2, _):
        visit(2 * k2, 0)
        visit(2 * k2 + 1, 1)
        return 0

    lax.fori_loop(0, NPAIR, pair, 0)
    out_drain(1)


@jax.jit
def _run(idx, token_embedding, position_embedding):
    mesh = plsc.VectorSubcoreMesh(core_axis_name="c", subcore_axis_name="s")
    k = pl.kernel(
        _sc_body,
        out_type=jax.ShapeDtypeStruct((B, EMBED), jnp.float32),
        mesh=mesh,
        scratch_types=[
            pltpu.VMEM((NUM_POS, EMBED), jnp.float32),
            pltpu.VMEM((ROWS_PER_W,), jnp.int32),
            pltpu.VMEM((CHUNK, EMBED), jnp.float32),
            pltpu.VMEM((CHUNK, EMBED), jnp.float32),
            pltpu.SemaphoreType.DMA,
            pltpu.SemaphoreType.DMA,
            pltpu.SemaphoreType.DMA,
            pltpu.SemaphoreType.DMA,
        ],
    )
    return k(token_embedding, idx, position_embedding)


def kernel(input_tokens, token_embedding, position_embedding):
    idx = input_tokens.reshape(-1).astype(jnp.int32)
    out = _run(idx, token_embedding, position_embedding)
    return out.reshape(BATCH, NUM_POS, EMBED)


# speedup vs baseline: 2.3074x; 1.9737x over previous
import jax
import jax.numpy as jnp
from jax import lax
from jax.experimental import pallas as pl
from jax.experimental.pallas import tpu as pltpu
from jax.experimental.pallas import tpu_sc as plsc

VOCAB = 49408
EMBED = 768
NUM_POS = 77
BATCH = 4096
B = BATCH * NUM_POS
NW = 32
ROWS_PER_W = B // NW
CHUNK = 64
NCHUNK = ROWS_PER_W // CHUNK
NPAIR = NCHUNK // 2

ELEMS_PER_TC_BLK = 8
TC_ROWS = ELEMS_PER_TC_BLK * NUM_POS


def _sc_body(tok_hbm, idx_hbm, out_hbm, idx_v, rows0, rows1,
             gsem0, gsem1, osem0, osem1):
    c = lax.axis_index("c")
    s = lax.axis_index("s")
    wid = s * 2 + c
    base = wid * ROWS_PER_W

    bufs = (rows0, rows1)
    gsems = (gsem0, gsem1)
    osems = (osem0, osem1)

    pltpu.sync_copy(idx_hbm.at[pl.ds(base, ROWS_PER_W)], idx_v)

    def gather_start(g, b):
        pltpu.async_copy(tok_hbm.at[idx_v.at[pl.ds(g * CHUNK, CHUNK)]],
                         bufs[b], gsems[b])

    def gather_wait(g, b):
        pltpu.make_async_copy(tok_hbm.at[idx_v.at[pl.ds(g * CHUNK, CHUNK)]],
                              bufs[b], gsems[b]).wait()

    def out_start(g, b):
        pltpu.async_copy(bufs[b], out_hbm.at[pl.ds(base + g * CHUNK, CHUNK)],
                         osems[b])

    def out_drain(b):
        pltpu.make_async_copy(bufs[b], out_hbm.at[pl.ds(base, CHUNK)],
                              osems[b]).wait()

    gather_start(0, 0)

    def visit(g, b):
        bn = 1 - b
        gather_wait(g, b)

        @pl.when(g >= 1)
        def _():
            out_drain(bn)

        @pl.when(g + 1 < NCHUNK)
        def _():
            gather_start(g + 1, bn)

        out_start(g, b)

    def pair(k2, _):
        visit(2 * k2, 0)
        visit(2 * k2 + 1, 1)
        return 0

    lax.fori_loop(0, NPAIR, pair, 0)
    out_drain(1)


def _tc_add_body(gath_ref, pos_ref, out_ref):
    summed = gath_ref[...] + pos_ref[...]
    for k in range(ELEMS_PER_TC_BLK):
        out_ref[k] = summed[k * NUM_POS:(k + 1) * NUM_POS, :]


@jax.jit
def _run(idx, token_embedding, pos_tiled):
    mesh = plsc.VectorSubcoreMesh(core_axis_name="c", subcore_axis_name="s")
    gather_k = pl.kernel(
        _sc_body,
        out_type=jax.ShapeDtypeStruct((B, EMBED), jnp.float32),
        mesh=mesh,
        scratch_types=[
            pltpu.VMEM((ROWS_PER_W,), jnp.int32),
            pltpu.VMEM((CHUNK, EMBED), jnp.float32),
            pltpu.VMEM((CHUNK, EMBED), jnp.float32),
            pltpu.SemaphoreType.DMA,
            pltpu.SemaphoreType.DMA,
            pltpu.SemaphoreType.DMA,
            pltpu.SemaphoreType.DMA,
        ],
    )
    gath = gather_k(token_embedding, idx)

    add_k = pl.pallas_call(
        _tc_add_body,
        out_shape=jax.ShapeDtypeStruct((BATCH, NUM_POS, EMBED), jnp.float32),
        grid=(B // TC_ROWS,),
        in_specs=[
            pl.BlockSpec((TC_ROWS, EMBED), lambda i: (i, 0)),
            pl.BlockSpec((TC_ROWS, EMBED), lambda i: (0, 0)),
        ],
        out_specs=pl.BlockSpec((ELEMS_PER_TC_BLK, NUM_POS, EMBED),
                               lambda i: (i, 0, 0)),
    )
    return add_k(gath, pos_tiled)


def kernel(input_tokens, token_embedding, position_embedding):
    idx = input_tokens.reshape(-1).astype(jnp.int32)
    pos_tiled = jnp.tile(position_embedding, (ELEMS_PER_TC_BLK, 1))
    return _run(idx, token_embedding, pos_tiled)
